# baseline (speedup 1.0000x reference)
import functools

import jax
import jax.numpy as jnp
from jax import lax
from jax.experimental import pallas as pl
from jax.experimental.pallas import tpu as pltpu
from jax.experimental.pallas import tpu_sc as plsc

SEQ_PAD = 64
NBUF = 4


@functools.lru_cache(maxsize=None)
def _make_gather(NS: int, S: int, V: int, D: int):
    info = plsc.get_sparse_core_info()
    nw = info.num_cores * info.num_subcores
    n = NS // nw
    lead = NBUF - 1
    n_outer = n // NBUF

    @functools.partial(
        pl.kernel,
        mesh=plsc.VectorSubcoreMesh(core_axis_name="c", subcore_axis_name="s"),
        out_type=jax.ShapeDtypeStruct((NS, S, D), jnp.float32),
        scratch_types=(
            [pltpu.VMEM((n, SEQ_PAD), jnp.int32),
             pltpu.VMEM((NBUF, SEQ_PAD, D), jnp.float32)]
            + [pltpu.SemaphoreType.DMA] * (2 * NBUF)
        ),
    )
    def gather_kernel(idx_hbm, table_hbm, out_hbm, idx_v, rows_v, *sems):
        gsem = sems[:NBUF]
        ssem = sems[NBUF:]
        wid = lax.axis_index("s") * info.num_cores + lax.axis_index("c")
        pltpu.sync_copy(idx_hbm.at[wid], idx_v)
        base = wid * n

        def gather_start(j, b):
            pltpu.async_copy(table_hbm.at[idx_v.at[j]], rows_v.at[b], gsem[b])

        def gather_wait(j, b):
            pltpu.make_async_copy(
                table_hbm.at[idx_v.at[j]], rows_v.at[b], gsem[b]).wait()

        def store_start(j, b):
            pltpu.async_copy(
                rows_v.at[b, pl.ds(0, S)], out_hbm.at[base + j], ssem[b])

        def store_wait(j, b):
            pltpu.make_async_copy(
                rows_v.at[b, pl.ds(0, S)], out_hbm.at[base + j],
                ssem[b]).wait()

        def step(j, b, first, last):
            bg = (b + lead) % NBUF
            if not last:
                if not first:
                    store_wait(j - 1, bg)
                gather_start(j + lead, bg)
            gather_wait(j, b)
            store_start(j, b)

        for m in range(lead):
            gather_start(m, m)
        for b in range(NBUF):
            step(b, b, first=(b == 0), last=False)

        def outer(j0, carry):
            for b in range(NBUF):
                step(j0 * NBUF + b, b, first=False, last=False)
            return carry

        lax.fori_loop(1, n_outer - 1, outer, 0)

        for b in range(NBUF):
            j = (n_outer - 1) * NBUF + b
            step(j, b, first=False, last=(j + lead >= n))
        for b in range(NBUF):
            store_wait(n - NBUF + b, b)

    return gather_kernel


def kernel(x, W):
    NS, S = x.shape
    fn = _make_gather(NS, S, W.shape[0], W.shape[1])
    info = plsc.get_sparse_core_info()
    nw = info.num_cores * info.num_subcores
    idx = jnp.pad(x.astype(jnp.int32), ((0, 0), (0, SEQ_PAD - S)))
    idx = idx.reshape(nw, NS // nw, SEQ_PAD)
    return fn(idx, W)


# speedup vs baseline: 14.8915x; 14.8915x over previous
import functools

import jax
import jax.numpy as jnp
from jax import lax
from jax.experimental import pallas as pl
from jax.experimental.pallas import tpu as pltpu
from jax.experimental.pallas import tpu_sc as plsc

SEQ_PAD = 56
NBUF = 4


@functools.lru_cache(maxsize=None)
def _make_gather(NS: int, S: int, V: int, D: int):
    info = plsc.get_sparse_core_info()
    nw = info.num_cores * info.num_subcores
    n = NS // nw
    lead = NBUF - 1
    n_outer = n // NBUF

    @functools.partial(
        pl.kernel,
        mesh=plsc.VectorSubcoreMesh(core_axis_name="c", subcore_axis_name="s"),
        out_type=jax.ShapeDtypeStruct((NS, S, D), jnp.float32),
        scratch_types=(
            [pltpu.VMEM((n, SEQ_PAD), jnp.int32),
             pltpu.VMEM((NBUF, SEQ_PAD, D), jnp.float32)]
            + [pltpu.SemaphoreType.DMA] * (2 * NBUF)
        ),
    )
    def gather_kernel(idx_hbm, table_hbm, out_hbm, idx_v, rows_v, *sems):
        gsem = sems[:NBUF]
        ssem = sems[NBUF:]
        wid = lax.axis_index("s") * info.num_cores + lax.axis_index("c")
        pltpu.sync_copy(idx_hbm.at[wid], idx_v)
        base = wid * n

        def gather_start(j, b):
            pltpu.async_copy(table_hbm.at[idx_v.at[j]], rows_v.at[b], gsem[b])

        def gather_wait(j, b):
            pltpu.make_async_copy(
                table_hbm.at[idx_v.at[j]], rows_v.at[b], gsem[b]).wait()

        def store_start(j, b):
            pltpu.async_copy(
                rows_v.at[b, pl.ds(0, S)], out_hbm.at[base + j], ssem[b])

        def store_wait(j, b):
            pltpu.make_async_copy(
                rows_v.at[b, pl.ds(0, S)], out_hbm.at[base + j],
                ssem[b]).wait()

        def step(j, b, first, last):
            bg = (b + lead) % NBUF
            if not last:
                if not first:
                    store_wait(j - 1, bg)
                gather_start(j + lead, bg)
            gather_wait(j, b)
            store_start(j, b)

        for m in range(lead):
            gather_start(m, m)
        for b in range(NBUF):
            step(b, b, first=(b == 0), last=False)

        def outer(j0, carry):
            for b in range(NBUF):
                step(j0 * NBUF + b, b, first=False, last=False)
            return carry

        lax.fori_loop(1, n_outer - 1, outer, 0)

        for b in range(NBUF):
            j = (n_outer - 1) * NBUF + b
            step(j, b, first=False, last=(j + lead >= n))
        for b in range(NBUF):
            store_wait(n - NBUF + b, b)

    return gather_kernel


def kernel(x, W):
    NS, S = x.shape
    fn = _make_gather(NS, S, W.shape[0], W.shape[1])
    info = plsc.get_sparse_core_info()
    nw = info.num_cores * info.num_subcores
    npad = SEQ_PAD - S
    pad = (jnp.arange(NS * npad, dtype=jnp.int32) * 97) % W.shape[0]
    idx = jnp.concatenate(
        [x.astype(jnp.int32), pad.reshape(NS, npad)], axis=1)
    idx = idx.reshape(nw, NS // nw, SEQ_PAD)
    return fn(idx, W)
